# Initial kernel scaffold; baseline (speedup 1.0000x reference)
import jax
import jax.numpy as jnp
from jax.experimental import pallas as pl


def kernel(x, edge_index, W1, b1, W2, b2, W_time, b_time, W_event, b_event):
    raise NotImplementedError("write your pallas kernel here")


# speedup vs baseline: 21.1373x; 21.1373x over previous
import functools

import jax
import jax.numpy as jnp
from jax import lax
from jax.experimental import pallas as pl
from jax.experimental.pallas import tpu as pltpu
from jax.experimental.pallas import tpu_sc as plsc

N_NODES = 10000
D_IN = 128
D_HID = 64

NC = 2
NS = 16
NW = NC * NS

CHUNK = 128
NP = 10240
RPT = NP // NS
DEGW = 16

_mesh = plsc.VectorSubcoreMesh(
    core_axis_name="c", subcore_axis_name="s", num_cores=NC, num_subcores=NS)


def _make_deg_kernel(K):
  @functools.partial(
      pl.kernel,
      out_type=jax.ShapeDtypeStruct((NC, NP, DEGW), jnp.float32),
      mesh=_mesh,
      scratch_types=[
          pltpu.VMEM((K, CHUNK), jnp.int32),
          pltpu.VMEM((CHUNK, DEGW), jnp.float32),
          pltpu.VMEM((RPT, DEGW), jnp.float32),
          pltpu.VMEM_SHARED((NP, DEGW), jnp.float32),
      ],
      compiler_params=pltpu.CompilerParams(use_tc_tiling_on_sc=False),
  )
  def deg_kernel(dst2d, ones16, zeros16, out, idx_v, ones_v, stage_v, deg_sh):
    cid = lax.axis_index("c")
    sid = lax.axis_index("s")
    wid = cid * NS + sid
    pltpu.sync_copy(zeros16, stage_v)
    pltpu.sync_copy(stage_v, deg_sh.at[pl.ds(sid * RPT, RPT)])
    pltpu.sync_copy(ones16, ones_v)
    pltpu.sync_copy(dst2d.at[wid], idx_v)
    plsc.subcore_barrier()

    @pl.loop(0, K)
    def _(j):
      pltpu.sync_copy(ones_v, deg_sh.at[idx_v.at[j]], add=True)

    plsc.subcore_barrier()
    pltpu.sync_copy(deg_sh.at[pl.ds(sid * RPT, RPT)], stage_v)
    pltpu.sync_copy(stage_v, out.at[cid].at[pl.ds(sid * RPT, RPT)])

  return deg_kernel


def _make_edge_kernel(K):
  @functools.partial(
      pl.kernel,
      out_type=jax.ShapeDtypeStruct((NC, NP, D_HID), jnp.float32),
      mesh=_mesh,
      scratch_types=[
          pltpu.VMEM((K, CHUNK), jnp.int32),
          pltpu.VMEM((K, CHUNK), jnp.int32),
          pltpu.VMEM((CHUNK, D_HID), jnp.float32),
          pltpu.VMEM((RPT, D_HID), jnp.float32),
          pltpu.VMEM_SHARED((NP, D_HID), jnp.float32),
          pltpu.SemaphoreType.DMA,
      ],
      compiler_params=pltpu.CompilerParams(use_tc_tiling_on_sc=False),
  )
  def edge_kernel(table, src2d, dst2d, zeros64, out, src_v, dst_v, buf,
                  stage_v, agg_sh, sem):
    cid = lax.axis_index("c")
    sid = lax.axis_index("s")
    wid = cid * NS + sid
    pltpu.sync_copy(zeros64, stage_v)
    pltpu.sync_copy(stage_v, agg_sh.at[pl.ds(sid * RPT, RPT)])
    pltpu.sync_copy(src2d.at[wid], src_v)
    pltpu.sync_copy(dst2d.at[wid], dst_v)
    plsc.subcore_barrier()

    @pl.loop(0, K)
    def _(j):
      pltpu.async_copy(table.at[src_v.at[j]], buf, sem).wait()
      pltpu.sync_copy(buf, agg_sh.at[dst_v.at[j]], add=True)

    plsc.subcore_barrier()
    pltpu.sync_copy(agg_sh.at[pl.ds(sid * RPT, RPT)], stage_v)
    pltpu.sync_copy(stage_v, out.at[cid].at[pl.ds(sid * RPT, RPT)])

  return edge_kernel


BR = 2048


def _tc_layer1(xp, W1, d0, d1):
  def body(x_ref, w_ref, d0_ref, d1_ref, scaled_ref, dinv_ref):
    dinv = lax.rsqrt(jnp.maximum(d0_ref[...] + d1_ref[...], 1.0))
    h = jnp.dot(x_ref[...], w_ref[...], preferred_element_type=jnp.float32)
    scaled_ref[...] = h * dinv
    dinv_ref[...] = dinv

  grid = (NP // BR,)
  return pl.pallas_call(
      body,
      grid=grid,
      in_specs=[
          pl.BlockSpec((BR, D_IN), lambda i: (i, 0)),
          pl.BlockSpec((D_IN, D_HID), lambda i: (0, 0)),
          pl.BlockSpec((BR, 1), lambda i: (i, 0)),
          pl.BlockSpec((BR, 1), lambda i: (i, 0)),
      ],
      out_specs=[
          pl.BlockSpec((BR, D_HID), lambda i: (i, 0)),
          pl.BlockSpec((BR, 1), lambda i: (i, 0)),
      ],
      out_shape=[
          jax.ShapeDtypeStruct((NP, D_HID), jnp.float32),
          jax.ShapeDtypeStruct((NP, 1), jnp.float32),
      ],
  )(xp, W1, d0, d1)


def _tc_layer2(a0, a1, dinv, W2, b1):
  def body(a0_ref, a1_ref, dinv_ref, w_ref, b_ref, out_ref):
    dinv = dinv_ref[...]
    h1 = jnp.maximum((a0_ref[...] + a1_ref[...]) * dinv + b_ref[...], 0.0)
    out_ref[...] = jnp.dot(
        h1, w_ref[...], preferred_element_type=jnp.float32) * dinv

  grid = (NP // BR,)
  return pl.pallas_call(
      body,
      grid=grid,
      in_specs=[
          pl.BlockSpec((BR, D_HID), lambda i: (i, 0)),
          pl.BlockSpec((BR, D_HID), lambda i: (i, 0)),
          pl.BlockSpec((BR, 1), lambda i: (i, 0)),
          pl.BlockSpec((D_HID, D_HID), lambda i: (0, 0)),
          pl.BlockSpec((1, D_HID), lambda i: (0, 0)),
      ],
      out_specs=pl.BlockSpec((BR, D_HID), lambda i: (i, 0)),
      out_shape=jax.ShapeDtypeStruct((NP, D_HID), jnp.float32),
  )(a0, a1, dinv, W2, b1)


def _tc_heads(a0, a1, dinv, b2, W_he, b_he):
  def body(a0_ref, a1_ref, dinv_ref, b2_ref, w_ref, bh_ref, out_ref):
    dinv = dinv_ref[...]
    h = jnp.maximum((a0_ref[...] + a1_ref[...]) * dinv + b2_ref[...], 0.0)
    out_ref[...] = jnp.dot(
        h, w_ref[...], preferred_element_type=jnp.float32) + bh_ref[...]

  grid = (NP // BR,)
  return pl.pallas_call(
      body,
      grid=grid,
      in_specs=[
          pl.BlockSpec((BR, D_HID), lambda i: (i, 0)),
          pl.BlockSpec((BR, D_HID), lambda i: (i, 0)),
          pl.BlockSpec((BR, 1), lambda i: (i, 0)),
          pl.BlockSpec((1, D_HID), lambda i: (0, 0)),
          pl.BlockSpec((D_HID, 2), lambda i: (0, 0)),
          pl.BlockSpec((1, 2), lambda i: (0, 0)),
      ],
      out_specs=pl.BlockSpec((BR, 2), lambda i: (i, 0)),
      out_shape=jax.ShapeDtypeStruct((NP, 2), jnp.float32),
  )(a0, a1, dinv, b2, W_he, b_he)


def kernel(x, edge_index, W1, b1, W2, b2, W_time, b_time, W_event, b_event):
  n_edges = edge_index.shape[1]
  e_tot = n_edges + N_NODES
  K = -(-e_tot // (NW * CHUNK))
  ep = K * NW * CHUNK

  src = edge_index[0]
  dst = edge_index[1]
  self_ix = jnp.arange(N_NODES, dtype=jnp.int32)
  pad = ep - e_tot
  srcp = jnp.concatenate(
      [src, self_ix, jnp.zeros((pad,), jnp.int32)]).reshape(NW, K, CHUNK)
  dstp = jnp.concatenate(
      [dst, self_ix,
       jnp.full((pad,), N_NODES, jnp.int32)]).reshape(NW, K, CHUNK)
  xp = jnp.concatenate(
      [x, jnp.zeros((NP - N_NODES, D_IN), jnp.float32)])

  ones16 = jnp.ones((CHUNK, DEGW), jnp.float32)
  zeros16 = jnp.zeros((RPT, DEGW), jnp.float32)
  zeros64 = jnp.zeros((RPT, D_HID), jnp.float32)

  degp = _make_deg_kernel(K)(dstp, ones16, zeros16)
  d0 = degp[0, :, :1]
  d1 = degp[1, :, :1]

  scaled1, dinv = _tc_layer1(xp, W1, d0, d1)

  edge_k = _make_edge_kernel(K)
  agg1 = edge_k(scaled1, srcp, dstp, zeros64)
  scaled2 = _tc_layer2(agg1[0], agg1[1], dinv, W2, b1.reshape(1, D_HID))

  agg2 = edge_k(scaled2, srcp, dstp, zeros64)
  W_he = jnp.concatenate([W_time, W_event], axis=1)
  b_he = jnp.concatenate([b_time, b_event]).reshape(1, 2)
  out = _tc_heads(agg2[0], agg2[1], dinv, b2.reshape(1, D_HID), W_he, b_he)
  return (out[:N_NODES, :1], out[:N_NODES, 1:2])
